# baseline (device time: 345169 ns/iter reference)
import jax
import jax.numpy as jnp
from jax import lax
from jax.experimental import pallas as pl
from jax.experimental.pallas import tpu as pltpu

NZ = 4
B, S, H, D = 1, 1024, 16, 128
HH = H // 2
SCALE = D ** -0.5


def kernel(Q, K, V):
    def body(q_ref, k_ref, v_ref, out_ref,
             bkf, bvf, bkb, bvb, ml_ref,
             fsend, frecv, bsend, brecv):
        my_x = lax.axis_index("x")
        my_y = lax.axis_index("y")
        my_z = lax.axis_index("z")
        left = (my_z - 1) % NZ
        right = (my_z + 1) % NZ

        barrier = pltpu.get_barrier_semaphore()
        for nbr in (left, right):
            pl.semaphore_signal(
                barrier, inc=1,
                device_id=(my_x, my_y, nbr),
                device_id_type=pl.DeviceIdType.MESH,
            )
        pl.semaphore_wait(barrier, 2)

        ones = jnp.ones((S, 128), jnp.bfloat16)

        def process(n_heads, h0, k_at, v_at, first=False):
            def head_body(i, carry):
                h = h0 + i
                s = lax.dot_general(
                    q_ref[h], k_at(i), (((1,), (0,)), ((), ())),
                    preferred_element_type=jnp.float32)
                p = jnp.exp(s.astype(jnp.bfloat16))
                lsum = lax.dot_general(
                    p, ones, (((1,), (0,)), ((), ())),
                    preferred_element_type=jnp.float32)
                pv = lax.dot_general(
                    p, v_at(i), (((1,), (0,)), ((), ())),
                    preferred_element_type=jnp.float32)
                if first:
                    ml_ref[h, :, 0:1] = lsum[:, 0:1]
                    out_ref[h] = pv
                else:
                    ml_ref[h, :, 0:1] += lsum[:, 0:1]
                    out_ref[h] = out_ref[h] + pv
                return carry

            lax.fori_loop(0, n_heads, head_body, 0, unroll=2)

        def start_hop(hop):
            if hop == 0:
                srcs = (k_ref.at[:HH], v_ref.at[:HH],
                        k_ref.at[HH:], v_ref.at[HH:])
            else:
                srcs = (bkf.at[hop - 1], bvf.at[hop - 1],
                        bkb.at[hop - 1], bvb.at[hop - 1])
            dsts = (bkf.at[hop], bvf.at[hop], bkb.at[hop], bvb.at[hop])
            sems = ((fsend, frecv), (fsend, frecv),
                    (bsend, brecv), (bsend, brecv))
            tgts = (right, right, left, left)
            rdmas = []
            for j in range(4):
                snd, rcv = sems[j]
                i = 2 * hop + (j % 2)
                r = pltpu.make_async_remote_copy(
                    src_ref=srcs[j], dst_ref=dsts[j],
                    send_sem=snd.at[i], recv_sem=rcv.at[i],
                    device_id=(my_x, my_y, tgts[j]),
                    device_id_type=pl.DeviceIdType.MESH,
                )
                r.start()
                rdmas.append(r)
            return rdmas

        def fwd_at(slot):
            return (lambda i, s_=slot: bkf[s_, i],
                    lambda i, s_=slot: bvf[s_, i])

        def bwd_at(slot):
            return (lambda i, s_=slot: bkb[s_, i],
                    lambda i, s_=slot: bvb[s_, i])

        rdmas = start_hop(0)
        process(H, 0, lambda i: k_ref[i], lambda i: v_ref[i], first=True)
        for r in rdmas:
            r.wait()

        for hop in range(1, NZ - 1):
            rdmas = start_hop(hop)
            kf, vf = fwd_at(hop - 1)
            kb_, vb_ = bwd_at(hop - 1)
            process(HH, 0, kf, vf)
            process(HH, HH, kb_, vb_)
            for r in rdmas:
                r.wait()
        kf, vf = fwd_at(NZ - 2)
        kb_, vb_ = bwd_at(NZ - 2)
        process(HH, 0, kf, vf)
        process(HH, HH, kb_, vb_)

        def norm_body(h, carry):
            out_ref[h] = out_ref[h] / ml_ref[h, :, 0:1]
            return carry

        lax.fori_loop(0, H, norm_body, 0)

    qb = (Q[0].transpose(1, 0, 2) * SCALE).astype(jnp.bfloat16)
    kb = K[0].transpose(1, 2, 0).astype(jnp.bfloat16)
    vb = V[0].transpose(1, 0, 2).astype(jnp.bfloat16)

    nsem = 2 * (NZ - 1)
    out = pl.pallas_call(
        body,
        out_shape=jax.ShapeDtypeStruct((H, S, D), jnp.float32),
        in_specs=[pl.BlockSpec(memory_space=pltpu.VMEM)] * 3,
        out_specs=pl.BlockSpec(memory_space=pltpu.VMEM),
        scratch_shapes=[
            pltpu.VMEM((NZ - 1, HH, D, S), jnp.bfloat16),
            pltpu.VMEM((NZ - 1, HH, S, D), jnp.bfloat16),
            pltpu.VMEM((NZ - 1, HH, D, S), jnp.bfloat16),
            pltpu.VMEM((NZ - 1, HH, S, D), jnp.bfloat16),
            pltpu.VMEM((H, S, 128), jnp.float32),
            pltpu.SemaphoreType.DMA((nsem,)),
            pltpu.SemaphoreType.DMA((nsem,)),
            pltpu.SemaphoreType.DMA((nsem,)),
            pltpu.SemaphoreType.DMA((nsem,)),
        ],
        compiler_params=pltpu.CompilerParams(
            collective_id=0,
            vmem_limit_bytes=100 * 1024 * 1024,
        ),
    )(qb, kb, vb)
    return out.transpose(1, 0, 2)[None]


# device time: 273188 ns/iter; 1.2635x vs baseline; 1.2635x over previous
import os

import jax
import jax.numpy as jnp
from jax import lax
from jax.experimental import pallas as pl
from jax.experimental.pallas import tpu as pltpu

_NO_COMM = bool(os.environ.get("NO_COMM"))

NZ = 4
B, S, H, D = 1, 1024, 16, 128
HH = H // 2
SCALE = D ** -0.5


def kernel(Q, K, V):
    def body(q_ref, k_ref, v_ref, out_ref,
             zk, zv, xk, xv, ml_ref,
             zsend, zrecv, xsend, xrecv):
        my_x = lax.axis_index("x")
        my_y = lax.axis_index("y")
        my_z = lax.axis_index("z")
        zleft = (my_z - 1) % NZ
        zright = (my_z + 1) % NZ
        h0_mine = HH * my_x
        h0_partner = HH - h0_mine

        if not _NO_COMM:
            barrier = pltpu.get_barrier_semaphore()
            for tgt in ((my_x, my_y, zleft), (1 - my_x, my_y, my_z)):
                pl.semaphore_signal(
                    barrier, inc=1, device_id=tgt,
                    device_id_type=pl.DeviceIdType.MESH,
                )
            pl.semaphore_wait(barrier, 2)

        ones = jnp.ones((S, 128), jnp.bfloat16)

        def process(n_heads, h0, k_at, v_at, first=False):
            def head_body(i, carry):
                h = h0 + i
                s = lax.dot_general(
                    q_ref[h], k_at(i), (((1,), (0,)), ((), ())),
                    preferred_element_type=jnp.float32)
                p = jnp.exp(s.astype(jnp.bfloat16))
                lsum = lax.dot_general(
                    p, ones, (((1,), (0,)), ((), ())),
                    preferred_element_type=jnp.float32)
                pv = lax.dot_general(
                    p, v_at(i), (((1,), (0,)), ((), ())),
                    preferred_element_type=jnp.float32)
                if first:
                    ml_ref[h, :, 0:1] = lsum[:, 0:1]
                    out_ref[h] = pv
                else:
                    ml_ref[h, :, 0:1] += lsum[:, 0:1]
                    out_ref[h] = out_ref[h] + pv
                return carry

            lax.fori_loop(0, n_heads, head_body, 0, unroll=2)

        def start_z(hop):
            if _NO_COMM:
                return []
            if hop == 0:
                srck = k_ref.at[pl.ds(h0_mine, HH)]
                srcv = v_ref.at[pl.ds(h0_mine, HH)]
            else:
                srck, srcv = zk.at[hop - 1], zv.at[hop - 1]
            rdmas = []
            for j, (src, dstbuf, snd, rcv) in enumerate(
                    ((srck, zk, zsend, zrecv), (srcv, zv, zsend, zrecv))):
                r = pltpu.make_async_remote_copy(
                    src_ref=src, dst_ref=dstbuf.at[hop],
                    send_sem=snd.at[2 * hop + j], recv_sem=rcv.at[2 * hop + j],
                    device_id=(my_x, my_y, zright),
                    device_id_type=pl.DeviceIdType.MESH,
                )
                r.start()
                rdmas.append(r)
            return rdmas

        def start_x(c):
            if _NO_COMM:
                return []
            rdmas = []
            for j, (srcbuf, dstbuf) in enumerate(((zk, xk), (zv, xv))):
                r = pltpu.make_async_remote_copy(
                    src_ref=srcbuf.at[c], dst_ref=dstbuf.at[c],
                    send_sem=xsend.at[2 * c + j], recv_sem=xrecv.at[2 * c + j],
                    device_id=(1 - my_x, my_y, my_z),
                    device_id_type=pl.DeviceIdType.MESH,
                )
                r.start()
                rdmas.append(r)
            return rdmas

        z_rdmas = start_z(0)
        process(H, 0, lambda i: k_ref[i], lambda i: v_ref[i], first=True)
        for r in z_rdmas:
            r.wait()

        for c in range(NZ - 1):
            z_rdmas = start_z(c + 1) if c < NZ - 2 else []
            x_rdmas = start_x(c)
            process(HH, h0_mine,
                    lambda i, c_=c: zk[c_, i], lambda i, c_=c: zv[c_, i])
            for r in x_rdmas:
                r.wait()
            process(HH, h0_partner,
                    lambda i, c_=c: xk[c_, i], lambda i, c_=c: xv[c_, i])
            for r in z_rdmas:
                r.wait()

        def norm_body(h, carry):
            out_ref[h] = out_ref[h] / ml_ref[h, :, 0:1]
            return carry

        lax.fori_loop(0, H, norm_body, 0)

    qb = (Q[0].transpose(1, 0, 2) * SCALE).astype(jnp.bfloat16)
    kb = K[0].transpose(1, 2, 0).astype(jnp.bfloat16)
    vb = V[0].transpose(1, 0, 2).astype(jnp.bfloat16)

    nsem = 2 * (NZ - 1)
    out = pl.pallas_call(
        body,
        out_shape=jax.ShapeDtypeStruct((H, S, D), jnp.float32),
        in_specs=[pl.BlockSpec(memory_space=pltpu.VMEM)] * 3,
        out_specs=pl.BlockSpec(memory_space=pltpu.VMEM),
        scratch_shapes=[
            pltpu.VMEM((NZ - 1, HH, D, S), jnp.bfloat16),
            pltpu.VMEM((NZ - 1, HH, S, D), jnp.bfloat16),
            pltpu.VMEM((NZ - 1, HH, D, S), jnp.bfloat16),
            pltpu.VMEM((NZ - 1, HH, S, D), jnp.bfloat16),
            pltpu.VMEM((H, S, 128), jnp.float32),
            pltpu.SemaphoreType.DMA((nsem,)),
            pltpu.SemaphoreType.DMA((nsem,)),
            pltpu.SemaphoreType.DMA((nsem,)),
            pltpu.SemaphoreType.DMA((nsem,)),
        ],
        compiler_params=pltpu.CompilerParams(
            collective_id=None if _NO_COMM else 0,
            vmem_limit_bytes=100 * 1024 * 1024,
        ),
    )(qb, kb, vb)
    return out.transpose(1, 0, 2)[None]


# device time: 208100 ns/iter; 1.6587x vs baseline; 1.3128x over previous
import os

import jax
import jax.numpy as jnp
from jax import lax
from jax.experimental import pallas as pl
from jax.experimental.pallas import tpu as pltpu

_NO_COMM = bool(os.environ.get("NO_COMM"))

NZ = 4
B, S, H, D = 1, 1024, 16, 128
HQ = H // 4
SCALE = D ** -0.5


def _plane_coords(t):
    x = t // 2
    y = (t // 2 + t) % 2
    return x, y


def kernel(Q, K, V):
    def body(q_ref, k_ref, v_ref, out_ref,
             zk, zv, lqk, lqv, rqk, rqv, dk, dv, ml_ref,
             zsend, zrecv, psend, precv):
        my_x = lax.axis_index("x")
        my_y = lax.axis_index("y")
        my_z = lax.axis_index("z")
        zleft = (my_z - 1) % NZ
        zright = (my_z + 1) % NZ
        my_p = jnp.where(my_x == 0, my_y, 3 - my_y)
        pl_right = _plane_coords((my_p + 1) % 4)
        pl_left = _plane_coords((my_p + 3) % 4)
        h0_mine = HQ * my_p

        if not _NO_COMM:
            barrier = pltpu.get_barrier_semaphore()
            for tgt in ((my_x, my_y, zleft),
                        (pl_left[0], pl_left[1], my_z),
                        (pl_right[0], pl_right[1], my_z)):
                pl.semaphore_signal(
                    barrier, inc=1, device_id=tgt,
                    device_id_type=pl.DeviceIdType.MESH,
                )
            pl.semaphore_wait(barrier, 3)

        ones = jnp.ones((S, 128), jnp.bfloat16)

        def process(n_heads, h0, k_at, v_at, first=False):
            def head_body(i, carry):
                h = h0 + i
                s = lax.dot_general(
                    q_ref[h], k_at(i), (((1,), (0,)), ((), ())),
                    preferred_element_type=jnp.float32)
                p = jnp.exp(s.astype(jnp.bfloat16))
                lsum = lax.dot_general(
                    p, ones, (((1,), (0,)), ((), ())),
                    preferred_element_type=jnp.float32)
                pv = lax.dot_general(
                    p, v_at(i), (((1,), (0,)), ((), ())),
                    preferred_element_type=jnp.float32)
                if first:
                    ml_ref[h, :, 0:1] = lsum[:, 0:1]
                    out_ref[h] = pv
                else:
                    ml_ref[h, :, 0:1] += lsum[:, 0:1]
                    out_ref[h] = out_ref[h] + pv
                return carry

            lax.fori_loop(0, n_heads, head_body, 0, unroll=2)

        def rdma(src, dst, sems, idx, tgt):
            r = pltpu.make_async_remote_copy(
                src_ref=src, dst_ref=dst,
                send_sem=sems[0].at[idx], recv_sem=sems[1].at[idx],
                device_id=tgt, device_id_type=pl.DeviceIdType.MESH,
            )
            r.start()
            return r

        z_tgt = (my_x, my_y, zright)
        l_tgt = (pl_left[0], pl_left[1], my_z)
        r_tgt = (pl_right[0], pl_right[1], my_z)
        zsems = (zsend, zrecv)
        psems = (psend, precv)

        def start_z(hop):
            if _NO_COMM:
                return []
            if hop == 0:
                srck = k_ref.at[pl.ds(h0_mine, HQ)]
                srcv = v_ref.at[pl.ds(h0_mine, HQ)]
            else:
                srck, srcv = zk.at[hop - 1], zv.at[hop - 1]
            return [rdma(srck, zk.at[hop], zsems, 2 * hop, z_tgt),
                    rdma(srcv, zv.at[hop], zsems, 2 * hop + 1, z_tgt)]

        z_rdmas = start_z(0)
        process(H, 0, lambda i: k_ref[i], lambda i: v_ref[i], first=True)
        for r in z_rdmas:
            r.wait()

        done = []
        for c in range(NZ - 1):
            z_rdmas = start_z(c + 1) if c < NZ - 2 else []
            if not _NO_COMM:
                j = 6 * c
                qsends = [
                    rdma(zk.at[c], lqk.at[c], psems, j + 0, r_tgt),
                    rdma(zv.at[c], lqv.at[c], psems, j + 1, r_tgt),
                    rdma(zk.at[c], rqk.at[c], psems, j + 2, l_tgt),
                    rdma(zv.at[c], rqv.at[c], psems, j + 3, l_tgt),
                ]
            process(HQ, h0_mine,
                    lambda i, c_=c: zk[c_, i], lambda i, c_=c: zv[c_, i])
            if not _NO_COMM:
                qsends[0].wait_recv()
                done.append(rdma(lqk.at[c], dk.at[c], psems, j + 4, r_tgt))
                qsends[3].wait_recv()
                done.append(rdma(rqv.at[c], dv.at[c], psems, j + 5, l_tgt))
                qsends[1].wait_recv()
                qsends[2].wait_recv()
                done.extend(qsends)
            h_left = HQ * ((my_p + 3) % 4)
            h_right = HQ * ((my_p + 1) % 4)
            h_diag = HQ * ((my_p + 2) % 4)
            process(HQ, h_left,
                    lambda i, c_=c: lqk[c_, i], lambda i, c_=c: lqv[c_, i])
            process(HQ, h_right,
                    lambda i, c_=c: rqk[c_, i], lambda i, c_=c: rqv[c_, i])
            if not _NO_COMM:
                done[6 * c + 0].wait_recv()
                done[6 * c + 1].wait_recv()
            process(HQ, h_diag,
                    lambda i, c_=c: dk[c_, i], lambda i, c_=c: dv[c_, i])
            for r in z_rdmas:
                r.wait()

        for r in done:
            r.wait_send()

        def norm_body(h, carry):
            out_ref[h] = out_ref[h] / ml_ref[h, :, 0:1]
            return carry

        lax.fori_loop(0, H, norm_body, 0)

    qb = (Q[0].transpose(1, 0, 2) * SCALE).astype(jnp.bfloat16)
    kb = K[0].transpose(1, 2, 0).astype(jnp.bfloat16)
    vb = V[0].transpose(1, 0, 2).astype(jnp.bfloat16)

    out = pl.pallas_call(
        body,
        out_shape=jax.ShapeDtypeStruct((H, S, D), jnp.float32),
        in_specs=[pl.BlockSpec(memory_space=pltpu.VMEM)] * 3,
        out_specs=pl.BlockSpec(memory_space=pltpu.VMEM),
        scratch_shapes=[
            pltpu.VMEM((NZ - 1, HQ, D, S), jnp.bfloat16),
            pltpu.VMEM((NZ - 1, HQ, S, D), jnp.bfloat16),
            pltpu.VMEM((NZ - 1, HQ, D, S), jnp.bfloat16),
            pltpu.VMEM((NZ - 1, HQ, S, D), jnp.bfloat16),
            pltpu.VMEM((NZ - 1, HQ, D, S), jnp.bfloat16),
            pltpu.VMEM((NZ - 1, HQ, S, D), jnp.bfloat16),
            pltpu.VMEM((NZ - 1, HQ, D, S), jnp.bfloat16),
            pltpu.VMEM((NZ - 1, HQ, S, D), jnp.bfloat16),
            pltpu.VMEM((H, S, 128), jnp.float32),
            pltpu.SemaphoreType.DMA((2 * (NZ - 1),)),
            pltpu.SemaphoreType.DMA((2 * (NZ - 1),)),
            pltpu.SemaphoreType.DMA((6 * (NZ - 1),)),
            pltpu.SemaphoreType.DMA((6 * (NZ - 1),)),
        ],
        compiler_params=pltpu.CompilerParams(
            collective_id=None if _NO_COMM else 0,
            vmem_limit_bytes=100 * 1024 * 1024,
        ),
    )(qb, kb, vb)
    return out.transpose(1, 0, 2)[None]


# device time: 193578 ns/iter; 1.7831x vs baseline; 1.0750x over previous
import os

import jax
import jax.numpy as jnp
from jax import lax
from jax.experimental import pallas as pl
from jax.experimental.pallas import tpu as pltpu

_NO_COMM = bool(os.environ.get("NO_COMM"))

NZ = 4
B, S, H, D = 1, 1024, 16, 128
HQ = H // 4
SCALE = D ** -0.5


def _plane_coords(t):
    x = t // 2
    y = (t // 2 + t) % 2
    return x, y


def kernel(Q, K, V):
    def body(q_ref, k_ref, v_ref, out_ref,
             zk, zv, lqk, lqv, rqk, rqv, dk, dv, ml_ref,
             zsend, zrecv, psend, precv):
        my_x = lax.axis_index("x")
        my_y = lax.axis_index("y")
        my_z = lax.axis_index("z")
        zleft = (my_z - 1) % NZ
        zright = (my_z + 1) % NZ
        my_p = jnp.where(my_x == 0, my_y, 3 - my_y)
        pl_right = _plane_coords((my_p + 1) % 4)
        pl_left = _plane_coords((my_p + 3) % 4)
        h0_mine = HQ * my_p

        if not _NO_COMM:
            barrier = pltpu.get_barrier_semaphore()
            for tgt in ((my_x, my_y, zleft),
                        (pl_left[0], pl_left[1], my_z),
                        (pl_right[0], pl_right[1], my_z)):
                pl.semaphore_signal(
                    barrier, inc=1, device_id=tgt,
                    device_id_type=pl.DeviceIdType.MESH,
                )
            pl.semaphore_wait(barrier, 3)

        ones = jnp.ones((S, 128), jnp.bfloat16)

        def process(n_heads, h0, k_at, v_at, first=False):
            def head_body(i, carry):
                h = h0 + i
                s = lax.dot_general(
                    q_ref[h], k_at(i), (((1,), (0,)), ((), ())),
                    preferred_element_type=jnp.float32)
                p = jnp.exp(s.astype(jnp.bfloat16))
                lsum = lax.dot_general(
                    p, ones, (((1,), (0,)), ((), ())),
                    preferred_element_type=jnp.float32)
                pv = lax.dot_general(
                    p, v_at(i), (((1,), (0,)), ((), ())),
                    preferred_element_type=jnp.float32)
                if first:
                    ml_ref[h, :, 0:1] = lsum[:, 0:1]
                    out_ref[h] = pv
                else:
                    ml_ref[h, :, 0:1] += lsum[:, 0:1]
                    out_ref[h] = out_ref[h] + pv
                return carry

            lax.fori_loop(0, n_heads, head_body, 0, unroll=2)

        def rdma(src, dst, sems, idx, tgt):
            r = pltpu.make_async_remote_copy(
                src_ref=src, dst_ref=dst,
                send_sem=sems[0].at[idx], recv_sem=sems[1].at[idx],
                device_id=tgt, device_id_type=pl.DeviceIdType.MESH,
            )
            r.start()
            return r

        z_tgt = (my_x, my_y, zright)
        l_tgt = (pl_left[0], pl_left[1], my_z)
        r_tgt = (pl_right[0], pl_right[1], my_z)
        zsems = (zsend, zrecv)
        psems = (psend, precv)

        def start_z(hop):
            if _NO_COMM:
                return []
            if hop == 0:
                srck = k_ref.at[pl.ds(h0_mine, HQ)]
                srcv = v_ref.at[pl.ds(h0_mine, HQ)]
            else:
                srck, srcv = zk.at[hop - 1], zv.at[hop - 1]
            return [rdma(srck, zk.at[hop], zsems, 2 * hop, z_tgt),
                    rdma(srcv, zv.at[hop], zsems, 2 * hop + 1, z_tgt)]

        z_rdmas = start_z(0)
        process(H, 0, lambda i: k_ref[i], lambda i: v_ref[i], first=True)
        for r in z_rdmas:
            r.wait()

        def start_qsends(c):
            if _NO_COMM:
                return []
            j = 6 * c
            return [
                rdma(zk.at[c], lqk.at[c], psems, j + 0, r_tgt),
                rdma(zv.at[c], lqv.at[c], psems, j + 1, r_tgt),
                rdma(zk.at[c], rqk.at[c], psems, j + 2, l_tgt),
                rdma(zv.at[c], rqv.at[c], psems, j + 3, l_tgt),
            ]

        done = []
        qs = start_qsends(0)
        for c in range(NZ - 1):
            z_rdmas = start_z(c + 1) if c < NZ - 2 else []
            process(HQ, h0_mine,
                    lambda i, c_=c: zk[c_, i], lambda i, c_=c: zv[c_, i])
            if not _NO_COMM:
                j = 6 * c
                qs[0].wait_recv()
                done.append(rdma(lqk.at[c], dk.at[c], psems, j + 4, r_tgt))
                qs[3].wait_recv()
                done.append(rdma(rqv.at[c], dv.at[c], psems, j + 5, l_tgt))
                qs[1].wait_recv()
                qs[2].wait_recv()
                done.extend(qs)
            h_left = HQ * ((my_p + 3) % 4)
            h_right = HQ * ((my_p + 1) % 4)
            h_diag = HQ * ((my_p + 2) % 4)
            process(HQ, h_left,
                    lambda i, c_=c: lqk[c_, i], lambda i, c_=c: lqv[c_, i])
            process(HQ, h_right,
                    lambda i, c_=c: rqk[c_, i], lambda i, c_=c: rqv[c_, i])
            if c < NZ - 2:
                for r in z_rdmas:
                    r.wait()
                qs = start_qsends(c + 1)
            if not _NO_COMM:
                done[6 * c + 0].wait_recv()
                done[6 * c + 1].wait_recv()
            process(HQ, h_diag,
                    lambda i, c_=c: dk[c_, i], lambda i, c_=c: dv[c_, i])

        for r in done:
            r.wait_send()

        def norm_body(h, carry):
            out_ref[h] = out_ref[h] / ml_ref[h, :, 0:1]
            return carry

        lax.fori_loop(0, H, norm_body, 0)

    qb = (Q[0].transpose(1, 0, 2) * SCALE).astype(jnp.bfloat16)
    kb = K[0].transpose(1, 2, 0).astype(jnp.bfloat16)
    vb = V[0].transpose(1, 0, 2).astype(jnp.bfloat16)

    out = pl.pallas_call(
        body,
        out_shape=jax.ShapeDtypeStruct((H, S, D), jnp.float32),
        in_specs=[pl.BlockSpec(memory_space=pltpu.VMEM)] * 3,
        out_specs=pl.BlockSpec(memory_space=pltpu.VMEM),
        scratch_shapes=[
            pltpu.VMEM((NZ - 1, HQ, D, S), jnp.bfloat16),
            pltpu.VMEM((NZ - 1, HQ, S, D), jnp.bfloat16),
            pltpu.VMEM((NZ - 1, HQ, D, S), jnp.bfloat16),
            pltpu.VMEM((NZ - 1, HQ, S, D), jnp.bfloat16),
            pltpu.VMEM((NZ - 1, HQ, D, S), jnp.bfloat16),
            pltpu.VMEM((NZ - 1, HQ, S, D), jnp.bfloat16),
            pltpu.VMEM((NZ - 1, HQ, D, S), jnp.bfloat16),
            pltpu.VMEM((NZ - 1, HQ, S, D), jnp.bfloat16),
            pltpu.VMEM((H, S, 128), jnp.float32),
            pltpu.SemaphoreType.DMA((2 * (NZ - 1),)),
            pltpu.SemaphoreType.DMA((2 * (NZ - 1),)),
            pltpu.SemaphoreType.DMA((6 * (NZ - 1),)),
            pltpu.SemaphoreType.DMA((6 * (NZ - 1),)),
        ],
        compiler_params=pltpu.CompilerParams(
            collective_id=None if _NO_COMM else 0,
            vmem_limit_bytes=100 * 1024 * 1024,
        ),
    )(qb, kb, vb)
    return out.transpose(1, 0, 2)[None]


# device time: 193458 ns/iter; 1.7842x vs baseline; 1.0006x over previous
import os

import jax
import jax.numpy as jnp
from jax import lax
from jax.experimental import pallas as pl
from jax.experimental.pallas import tpu as pltpu

_NO_COMM = bool(os.environ.get("NO_COMM"))

NZ = 4
B, S, H, D = 1, 1024, 16, 128
HQ = H // 4
SCALE = D ** -0.5


def _plane_coords(t):
    x = t // 2
    y = (t // 2 + t) % 2
    return x, y


def kernel(Q, K, V):
    def body(q_ref, k_ref, v_ref, out_ref,
             zk, zv, lqk, lqv, rqk, rqv, dk, dv, ml_ref,
             zsend, zrecv, psend, precv):
        my_x = lax.axis_index("x")
        my_y = lax.axis_index("y")
        my_z = lax.axis_index("z")
        zleft = (my_z - 1) % NZ
        zright = (my_z + 1) % NZ
        my_p = jnp.where(my_x == 0, my_y, 3 - my_y)
        pl_right = _plane_coords((my_p + 1) % 4)
        pl_left = _plane_coords((my_p + 3) % 4)
        h0_mine = HQ * my_p

        if not _NO_COMM:
            barrier = pltpu.get_barrier_semaphore()
            for tgt in ((my_x, my_y, zleft),
                        (pl_left[0], pl_left[1], my_z),
                        (pl_right[0], pl_right[1], my_z)):
                pl.semaphore_signal(
                    barrier, inc=1, device_id=tgt,
                    device_id_type=pl.DeviceIdType.MESH,
                )
            pl.semaphore_wait(barrier, 3)

        ones = jnp.ones((S, 128), jnp.bfloat16)

        def process(n_heads, h0, k_at, v_at, first=False):
            def head_body(i, carry):
                h = h0 + i
                s = lax.dot_general(
                    q_ref[h], k_at(i), (((1,), (0,)), ((), ())),
                    preferred_element_type=jnp.float32)
                p = jnp.exp(s.astype(jnp.bfloat16))
                lsum = lax.dot_general(
                    p, ones, (((1,), (0,)), ((), ())),
                    preferred_element_type=jnp.float32)
                pv = lax.dot_general(
                    p, v_at(i), (((1,), (0,)), ((), ())),
                    preferred_element_type=jnp.float32)
                if first:
                    ml_ref[h, :, 0:1] = lsum[:, 0:1]
                    out_ref[h] = pv
                else:
                    ml_ref[h, :, 0:1] += lsum[:, 0:1]
                    out_ref[h] = out_ref[h] + pv
                return carry

            lax.fori_loop(0, n_heads, head_body, 0, unroll=2)

        def rdma(src, dst, sems, idx, tgt):
            r = pltpu.make_async_remote_copy(
                src_ref=src, dst_ref=dst,
                send_sem=sems[0].at[idx], recv_sem=sems[1].at[idx],
                device_id=tgt, device_id_type=pl.DeviceIdType.MESH,
            )
            r.start()
            return r

        z_tgt = (my_x, my_y, zright)
        l_tgt = (pl_left[0], pl_left[1], my_z)
        r_tgt = (pl_right[0], pl_right[1], my_z)
        zsems = (zsend, zrecv)
        psems = (psend, precv)

        def start_z(hop):
            if _NO_COMM:
                return []
            if hop == 0:
                srck = k_ref.at[pl.ds(h0_mine, HQ)]
                srcv = v_ref.at[pl.ds(h0_mine, HQ)]
            else:
                srck, srcv = zk.at[hop - 1], zv.at[hop - 1]
            return [rdma(srck, zk.at[hop], zsems, 2 * hop, z_tgt),
                    rdma(srcv, zv.at[hop], zsems, 2 * hop + 1, z_tgt)]

        z_rdmas = start_z(0)
        process(H, 0, lambda i: k_ref[i], lambda i: v_ref[i], first=True)

        def start_qsends_k(c):
            return [rdma(zk.at[c], lqk.at[c], psems, 6 * c + 0, r_tgt),
                    rdma(zk.at[c], rqk.at[c], psems, 6 * c + 2, l_tgt)]

        def start_qsends_v(c):
            return [rdma(zv.at[c], lqv.at[c], psems, 6 * c + 1, r_tgt),
                    rdma(zv.at[c], rqv.at[c], psems, 6 * c + 3, l_tgt)]

        done = []
        if not _NO_COMM:
            z_rdmas[0].wait_recv()
            qsk = start_qsends_k(0)
            z_rdmas[1].wait_recv()
            qsv = start_qsends_v(0)
            done.extend(z_rdmas)
        for c in range(NZ - 1):
            z_rdmas = start_z(c + 1) if c < NZ - 2 else []
            process(HQ, h0_mine,
                    lambda i, c_=c: zk[c_, i], lambda i, c_=c: zv[c_, i])
            fk = fv = None
            if not _NO_COMM:
                j = 6 * c
                qsk[0].wait_recv()
                fk = rdma(lqk.at[c], dk.at[c], psems, j + 4, r_tgt)
                qsv[1].wait_recv()
                fv = rdma(rqv.at[c], dv.at[c], psems, j + 5, l_tgt)
                qsv[0].wait_recv()
                qsk[1].wait_recv()
                done.extend(qsk + qsv + [fk, fv])
            h_left = HQ * ((my_p + 3) % 4)
            h_right = HQ * ((my_p + 1) % 4)
            h_diag = HQ * ((my_p + 2) % 4)
            process(HQ, h_left,
                    lambda i, c_=c: lqk[c_, i], lambda i, c_=c: lqv[c_, i])
            process(HQ, h_right,
                    lambda i, c_=c: rqk[c_, i], lambda i, c_=c: rqv[c_, i])
            if c < NZ - 2 and not _NO_COMM:
                z_rdmas[0].wait_recv()
                qsk = start_qsends_k(c + 1)
                z_rdmas[1].wait_recv()
                qsv = start_qsends_v(c + 1)
                done.extend(z_rdmas)
            if not _NO_COMM:
                fk.wait_recv()
                fv.wait_recv()
            process(HQ, h_diag,
                    lambda i, c_=c: dk[c_, i], lambda i, c_=c: dv[c_, i])

        for r in done:
            r.wait_send()

        def norm_body(h, carry):
            out_ref[h] = out_ref[h] / ml_ref[h, :, 0:1]
            return carry

        lax.fori_loop(0, H, norm_body, 0)

    qb = (Q[0].transpose(1, 0, 2) * SCALE).astype(jnp.bfloat16)
    kb = K[0].transpose(1, 2, 0).astype(jnp.bfloat16)
    vb = V[0].transpose(1, 0, 2).astype(jnp.bfloat16)

    out = pl.pallas_call(
        body,
        out_shape=jax.ShapeDtypeStruct((H, S, D), jnp.float32),
        in_specs=[pl.BlockSpec(memory_space=pltpu.VMEM)] * 3,
        out_specs=pl.BlockSpec(memory_space=pltpu.VMEM),
        scratch_shapes=[
            pltpu.VMEM((NZ - 1, HQ, D, S), jnp.bfloat16),
            pltpu.VMEM((NZ - 1, HQ, S, D), jnp.bfloat16),
            pltpu.VMEM((NZ - 1, HQ, D, S), jnp.bfloat16),
            pltpu.VMEM((NZ - 1, HQ, S, D), jnp.bfloat16),
            pltpu.VMEM((NZ - 1, HQ, D, S), jnp.bfloat16),
            pltpu.VMEM((NZ - 1, HQ, S, D), jnp.bfloat16),
            pltpu.VMEM((NZ - 1, HQ, D, S), jnp.bfloat16),
            pltpu.VMEM((NZ - 1, HQ, S, D), jnp.bfloat16),
            pltpu.VMEM((H, S, 128), jnp.float32),
            pltpu.SemaphoreType.DMA((2 * (NZ - 1),)),
            pltpu.SemaphoreType.DMA((2 * (NZ - 1),)),
            pltpu.SemaphoreType.DMA((6 * (NZ - 1),)),
            pltpu.SemaphoreType.DMA((6 * (NZ - 1),)),
        ],
        compiler_params=pltpu.CompilerParams(
            collective_id=None if _NO_COMM else 0,
            vmem_limit_bytes=100 * 1024 * 1024,
        ),
    )(qb, kb, vb)
    return out.transpose(1, 0, 2)[None]
